# baseline (speedup 1.0000x reference)
import jax
import jax.numpy as jnp
from jax.experimental import pallas as pl

_R_BLK = 8192


def _add_kernel(x_ref, fe_ref, o_ref):
    o_ref[...] = x_ref[...] + fe_ref[...]


def kernel(x, emb_table):
    b, c, f, t = x.shape
    fecol = emb_table[:f].T.reshape(c * f, 1)
    x2 = x.reshape(b * c * f, t)
    n_blk = (b * c * f) // _R_BLK
    per_b = (c * f) // _R_BLK
    out = pl.pallas_call(
        _add_kernel,
        grid=(n_blk,),
        in_specs=[
            pl.BlockSpec((_R_BLK, t), lambda i: (i, 0)),
            pl.BlockSpec((_R_BLK, 1), lambda i: (i % per_b, 0)),
        ],
        out_specs=pl.BlockSpec((_R_BLK, t), lambda i: (i, 0)),
        out_shape=jax.ShapeDtypeStruct((b * c * f, t), x.dtype),
    )(x2, fecol)
    return out.reshape(b, c, f, t)


# speedup vs baseline: 1.2667x; 1.2667x over previous
import jax
import jax.numpy as jnp
from jax.experimental import pallas as pl

_C_BLK = 64


def _add_kernel(x_ref, emb_ref, o_ref):
    j = pl.program_id(1)
    fe = emb_ref[...].T
    fe_half = jnp.where(j == 0, fe[:_C_BLK], fe[_C_BLK:])
    o_ref[...] = x_ref[...] + fe_half[None, :, :, None]


def kernel(x, emb_table):
    b, c, f, t = x.shape
    grid = (b, c // _C_BLK)
    return pl.pallas_call(
        _add_kernel,
        grid=grid,
        in_specs=[
            pl.BlockSpec((1, _C_BLK, f, t), lambda i, j: (i, j, 0, 0)),
            pl.BlockSpec((f, c), lambda i, j: (0, 0)),
        ],
        out_specs=pl.BlockSpec((1, _C_BLK, f, t), lambda i, j: (i, j, 0, 0)),
        out_shape=jax.ShapeDtypeStruct(x.shape, x.dtype),
    )(x, emb_table)
